# baseline (speedup 1.0000x reference)
import functools

import jax
import jax.numpy as jnp
from jax import lax
from jax.experimental import pallas as pl
from jax.experimental.pallas import tpu as pltpu
from jax.experimental.pallas import tpu_sc as plsc

R = 8192
S = 64
LANES = 16
CHUNKS = S // LANES
NC = 2
NS = 16
NW = NC * NS
RPW = R // NW
LAMBDA_OPACITY = 0.001
LAMBDA_DISTORTION = 0.001

_mesh = plsc.VectorSubcoreMesh(core_axis_name="c", subcore_axis_name="s")


@functools.partial(
    pl.kernel,
    out_type=jax.ShapeDtypeStruct((R,), jnp.float32),
    mesh=_mesh,
    scratch_types=[
        pltpu.VMEM((RPW * S,), jnp.float32),
        pltpu.VMEM((RPW * S,), jnp.float32),
        pltpu.VMEM((RPW * S,), jnp.float32),
        pltpu.VMEM((RPW,), jnp.float32),
        pltpu.SemaphoreType.DMA,
        pltpu.SemaphoreType.DMA,
    ],
    compiler_params=pltpu.CompilerParams(
        needs_layout_passes=False, skip_device_barrier=True
    ),
)
def _distortion_sc(ws_hbm, deltas_hbm, ts_hbm, out_hbm, w_v, d_v, t_v, out_v,
                   sem0, sem1):
    wid = lax.axis_index("s") * NC + lax.axis_index("c")
    base = wid * (RPW * S)
    half = (RPW // 2) * S
    cps0 = [
        pltpu.make_async_copy(src.at[pl.ds(base, half)], dst.at[pl.ds(0, half)], sem0)
        for src, dst in ((ws_hbm, w_v), (deltas_hbm, d_v), (ts_hbm, t_v))
    ]
    cps1 = [
        pltpu.make_async_copy(
            src.at[pl.ds(base + half, half)], dst.at[pl.ds(half, half)], sem1
        )
        for src, dst in ((ws_hbm, w_v), (deltas_hbm, d_v), (ts_hbm, t_v))
    ]
    for cp in cps0:
        cp.start()
    for cp in cps1:
        cp.start()
    for cp in cps0:
        cp.wait()

    lane15 = lax.iota(jnp.int32, LANES) == (LANES - 1)

    def ray_work(r):
        idx15 = jnp.full((LANES,), LANES - 1, jnp.int32)
        ws_c = []
        ts_c = []
        cws = []
        for c in range(CHUNKS):
            off = r * S + c * LANES
            w = w_v[pl.ds(off, LANES)]
            t = t_v[pl.ds(off, LANES)]
            ws_c.append(w)
            ts_c.append(t)
            cws.append(plsc.cumsum(w))
        acc1 = jnp.zeros((LANES,), jnp.float32)
        acc2 = jnp.zeros((LANES,), jnp.float32)
        acc3 = jnp.zeros((LANES,), jnp.float32)
        accu = jnp.zeros((LANES,), jnp.float32)
        cw_carry = jnp.zeros((LANES,), jnp.float32)
        for c in range(CHUNKS):
            off = r * S + c * LANES
            w = ws_c[c]
            t = ts_c[c]
            d = d_v[pl.ds(off, LANES)]
            wt = w * t
            w_excl = (cws[c] - w) + cw_carry
            acc1 = acc1 + wt * w_excl
            acc2 = acc2 + wt * w
            acc3 = acc3 + wt
            accu = accu + (w * w) * d
            cw_carry = cw_carry + cws[c][idx15]
        final = (4.0 * acc1 + 2.0 * acc2 + (1.0 / 3.0) * accu
                 - (2.0 * cw_carry) * acc3)
        tot = plsc.cumsum(final) * LAMBDA_DISTORTION
        idx = jnp.full((LANES,), r, jnp.int32)
        plsc.store_scatter(out_v, [idx], tot, mask=lane15)

    plsc.parallel_loop(0, RPW // 2, 1, unroll=2)(ray_work)
    for cp in cps1:
        cp.wait()
    plsc.parallel_loop(RPW // 2, RPW, 1, unroll=2)(ray_work)
    pltpu.sync_copy(out_v, out_hbm.at[pl.ds(wid * RPW, RPW)])


def _rgb_opacity_tc(rgb_p_ref, rgb_t_ref, op_ref, drgb_ref, dop_ref):
    diff = rgb_p_ref[...] - rgb_t_ref[...]
    drgb_ref[...] = diff * diff
    o = op_ref[...] + 1e-10
    dop_ref[...] = (-LAMBDA_OPACITY) * o * jnp.log(o)


def kernel(rgb_pred, rgb_target, opacity, ws, deltas, ts, rays_a):
    drgb_t, dop_t = pl.pallas_call(
        _rgb_opacity_tc,
        out_shape=(
            jax.ShapeDtypeStruct((3, R), jnp.float32),
            jax.ShapeDtypeStruct((1, R), jnp.float32),
        ),
    )(rgb_pred.T, rgb_target.T, opacity.T)
    d_distortion = _distortion_sc(ws, deltas, ts)
    return (drgb_t.T, dop_t.T, d_distortion)


# speedup vs baseline: 1.0617x; 1.0617x over previous
import functools

import jax
import jax.numpy as jnp
from jax import lax
from jax.experimental import pallas as pl
from jax.experimental.pallas import tpu as pltpu
from jax.experimental.pallas import tpu_sc as plsc

R = 8192
S = 64
LANES = 16
CHUNKS = S // LANES
NC = 2
NS = 16
NW = NC * NS
RPW = R // NW
LAMBDA_OPACITY = 0.001
LAMBDA_DISTORTION = 0.001

_mesh = plsc.VectorSubcoreMesh(core_axis_name="c", subcore_axis_name="s")


@functools.partial(
    pl.kernel,
    out_type=jax.ShapeDtypeStruct((R,), jnp.float32),
    mesh=_mesh,
    scratch_types=[
        pltpu.VMEM((RPW * S,), jnp.float32),
        pltpu.VMEM((RPW * S,), jnp.float32),
        pltpu.VMEM((RPW * S,), jnp.float32),
        pltpu.VMEM((RPW,), jnp.float32),
        pltpu.SemaphoreType.DMA,
        pltpu.SemaphoreType.DMA,
    ],
    compiler_params=pltpu.CompilerParams(
        needs_layout_passes=False, skip_device_barrier=True
    ),
)
def _distortion_sc(ws_hbm, deltas_hbm, ts_hbm, out_hbm, w_v, d_v, t_v, out_v,
                   sem0, sem1):
    wid = lax.axis_index("s") * NC + lax.axis_index("c")
    base = wid * (RPW * S)
    half = (RPW // 2) * S
    cps0 = [
        pltpu.make_async_copy(src.at[pl.ds(base, half)], dst.at[pl.ds(0, half)], sem0)
        for src, dst in ((ws_hbm, w_v), (deltas_hbm, d_v), (ts_hbm, t_v))
    ]
    cps1 = [
        pltpu.make_async_copy(
            src.at[pl.ds(base + half, half)], dst.at[pl.ds(half, half)], sem1
        )
        for src, dst in ((ws_hbm, w_v), (deltas_hbm, d_v), (ts_hbm, t_v))
    ]
    for cp in cps0:
        cp.start()
    for cp in cps1:
        cp.start()
    for cp in cps0:
        cp.wait()

    lane15 = lax.iota(jnp.int32, LANES) == (LANES - 1)

    def ray_work(r):
        idx15 = jnp.full((LANES,), LANES - 1, jnp.int32)
        ws_c = []
        ts_c = []
        cws = []
        for c in range(CHUNKS):
            off = r * S + c * LANES
            w = w_v[pl.ds(off, LANES)]
            t = t_v[pl.ds(off, LANES)]
            ws_c.append(w)
            ts_c.append(t)
            cws.append(plsc.cumsum(w))
        acc1 = jnp.zeros((LANES,), jnp.float32)
        acc2 = jnp.zeros((LANES,), jnp.float32)
        acc3 = jnp.zeros((LANES,), jnp.float32)
        accu = jnp.zeros((LANES,), jnp.float32)
        cw_carry = jnp.zeros((LANES,), jnp.float32)
        for c in range(CHUNKS):
            off = r * S + c * LANES
            w = ws_c[c]
            t = ts_c[c]
            d = d_v[pl.ds(off, LANES)]
            wt = w * t
            w_excl = (cws[c] - w) + cw_carry
            acc1 = acc1 + wt * w_excl
            acc2 = acc2 + wt * w
            acc3 = acc3 + wt
            accu = accu + (w * w) * d
            cw_carry = cw_carry + cws[c][idx15]
        final = (4.0 * acc1 + 2.0 * acc2 + (1.0 / 3.0) * accu
                 - (2.0 * cw_carry) * acc3)
        tot = plsc.cumsum(final) * LAMBDA_DISTORTION
        idx = jnp.full((LANES,), r, jnp.int32)
        plsc.store_scatter(out_v, [idx], tot, mask=lane15)

    plsc.parallel_loop(0, RPW // 2, 1)(ray_work)
    for cp in cps1:
        cp.wait()
    plsc.parallel_loop(RPW // 2, RPW, 1)(ray_work)
    pltpu.sync_copy(out_v, out_hbm.at[pl.ds(wid * RPW, RPW)])


def _rgb_opacity_tc(rgb_p_ref, rgb_t_ref, op_ref, drgb_ref, dop_ref):
    diff = rgb_p_ref[...] - rgb_t_ref[...]
    drgb_ref[...] = diff * diff
    o = op_ref[...] + 1e-10
    dop_ref[...] = (-LAMBDA_OPACITY) * o * jnp.log(o)


def kernel(rgb_pred, rgb_target, opacity, ws, deltas, ts, rays_a):
    drgb_t, dop_t = pl.pallas_call(
        _rgb_opacity_tc,
        out_shape=(
            jax.ShapeDtypeStruct((3, R), jnp.float32),
            jax.ShapeDtypeStruct((1, R), jnp.float32),
        ),
    )(rgb_pred.T, rgb_target.T, opacity.T)
    d_distortion = _distortion_sc(ws, deltas, ts)
    return (drgb_t.T, dop_t.T, d_distortion)
